# baseline (speedup 1.0000x reference)
import functools
import math

import jax
import jax.numpy as jnp
from jax import lax
from jax.experimental import pallas as pl
from jax.experimental.pallas import tpu as pltpu
from jax.experimental.pallas import tpu_sc as plsc

_D = 64
_SCALE = math.sqrt(_D)
_C = 128
_L = 16
_VB = 2048


def _relayout_block(t_ref, out_ref):
    x = t_ref[...]
    eye = jnp.eye(_D, dtype=jnp.float32) * jnp.float32(_SCALE)
    y = lax.dot_general(x, eye, (((0,), (0,)), ((), ())),
                        preferred_element_type=jnp.float32)
    y3 = y.reshape(_VB // 2, 2, _D)
    out_ref[...] = jnp.concatenate([y3[:, 0, :], y3[:, 1, :]], axis=1)


@functools.lru_cache(maxsize=None)
def _make_relayout(v: int):
    grid = (v + _VB - 1) // _VB
    return pl.pallas_call(
        _relayout_block,
        grid=(grid,),
        in_specs=[pl.BlockSpec((_D, _VB), lambda i: (0, i))],
        out_specs=pl.BlockSpec((_VB // 2, 2 * _D), lambda i: (i, 0)),
        out_shape=jax.ShapeDtypeStruct((v // 2, 2 * _D), jnp.float32),
    )


@functools.lru_cache(maxsize=None)
def _make_gather(S: int, B: int):
    info = plsc.get_sparse_core_info()
    nw = info.num_cores * info.num_subcores
    assert B == nw * _C

    mesh = plsc.VectorSubcoreMesh(core_axis_name="c", subcore_axis_name="s")
    ngrp = _C // _L

    @functools.partial(
        pl.kernel,
        mesh=mesh,
        out_type=jax.ShapeDtypeStruct((B, S, _D), jnp.float32),
        compiler_params=pltpu.CompilerParams(
            use_tc_tiling_on_sc=True, needs_layout_passes=False
        ),
        scratch_types=[
            pltpu.VMEM((S, _C), jnp.int32),
            pltpu.VMEM((_C,), jnp.int32),
            pltpu.VMEM((_C,), jnp.int32),
            pltpu.VMEM((_C + _L,), jnp.int32),
            pltpu.VMEM((_C + _L,), jnp.int32),
            pltpu.VMEM((_C, _C), jnp.float32),
            pltpu.VMEM((_C, _C), jnp.float32),
            pltpu.VMEM((_C, _D), jnp.float32),
            pltpu.VMEM((_C, _D), jnp.float32),
            pltpu.SemaphoreType.DMA,
            pltpu.SemaphoreType.DMA,
            pltpu.SemaphoreType.DMA,
            pltpu.SemaphoreType.DMA,
        ],
    )
    def k(tokens_hbm, table_hbm, out_hbm, tokbuf, idx0, idx1, par0, par1,
          g0, g1, o0, o1, gsem0, gsem1, osem0, osem1):
        idx = (idx0, idx1)
        par = (par0, par1)
        gbuf = (g0, g1)
        obuf = (o0, o1)
        gsem = (gsem0, gsem1)
        osem = (osem0, osem1)

        wid = lax.axis_index("s") * info.num_cores + lax.axis_index("c")
        col = wid * _C
        pltpu.sync_copy(tokens_hbm.at[:, pl.ds(col, _C)], tokbuf)

        def build(s, slot):
            for g in range(ngrp):
                sl = pl.ds(g * _L, _L)
                t = tokbuf[s, sl]
                idx[slot][sl] = lax.shift_right_logical(t, 1)
                par[slot][sl] = lax.shift_left(t & 1, 6)

        def gather(slot):
            return pltpu.async_copy(table_hbm.at[idx[slot]], gbuf[slot],
                                    gsem[slot])

        def out_slice(s):
            return out_hbm.at[pl.ds(col, _C), s, :]

        build(0, 0)
        gather(0)

        @pl.loop(0, S // 2)
        def _outer(so):
            for slot in range(2):
                s = so * 2 + slot
                nslot = 1 - slot

                @pl.when(s + 1 < S)
                def _prefetch():
                    build(s + 1, nslot)
                    gather(nslot)

                pltpu.make_async_copy(table_hbm.at[idx[slot]], gbuf[slot],
                                      gsem[slot]).wait()

                @pl.when(s >= 2)
                def _drain():
                    pltpu.make_async_copy(obuf[slot], out_slice(s - 2),
                                          osem[slot]).wait()

                src = gbuf[slot]
                dst = obuf[slot]
                pslot = par[slot]

                @plsc.parallel_loop(0, _C, unroll=4)
                def _select(j):
                    p = pslot[pl.ds(j, _L)][0]
                    for q in range(_D // _L):
                        dst[j, pl.ds(q * _L, _L)] = src[j, pl.ds(p + q * _L, _L)]

                pltpu.async_copy(dst, out_slice(s), osem[slot])

        pltpu.make_async_copy(obuf[0], out_slice(S - 2), osem[0]).wait()
        pltpu.make_async_copy(obuf[1], out_slice(S - 1), osem[1]).wait()

    return k


def kernel(tokens, table):
    s0, s1 = tokens.shape
    v, d = table.shape
    assert d == _D and v % 2 == 0
    tokens_t = tokens.T.astype(jnp.int32)
    table2 = _make_relayout(v)(table.T)
    return _make_gather(s1, s0)(tokens_t, table2)


# speedup vs baseline: 1.1235x; 1.1235x over previous
import functools
import math

import numpy as np
import jax
import jax.numpy as jnp
from jax import lax
from jax.experimental import pallas as pl
from jax.experimental.pallas import tpu as pltpu
from jax.experimental.pallas import tpu_sc as plsc

_D = 64
_SCALE = math.sqrt(_D)
_C = 128
_L = 16
_VB = 2048
_H = _VB // 2


def _relayout_block(t_ref, out_ref):
    x = t_ref[...]
    ii = lax.broadcasted_iota(jnp.int32, (_D, 2 * _D), 0)
    jj = lax.broadcasted_iota(jnp.int32, (_D, 2 * _D), 1)
    e1 = (ii == jj).astype(jnp.float32)
    e2 = ((ii + _D) == jj).astype(jnp.float32)
    dims = (((0,), (0,)), ((), ()))
    left = lax.dot_general(x[:, :_H], e1, dims,
                           preferred_element_type=jnp.float32)
    right = lax.dot_general(x[:, _H:], e2, dims,
                            preferred_element_type=jnp.float32)
    out_ref[...] = left + right


@functools.lru_cache(maxsize=None)
def _make_relayout(v: int):
    nb = (v + _VB - 1) // _VB
    return pl.pallas_call(
        _relayout_block,
        grid=(nb,),
        in_specs=[pl.BlockSpec((_D, _VB), lambda i: (0, i))],
        out_specs=pl.BlockSpec((_H, 2 * _D), lambda i: (i, 0)),
        out_shape=jax.ShapeDtypeStruct((nb * _H, 2 * _D), jnp.float32),
    )


@functools.lru_cache(maxsize=None)
def _make_gather(S: int, B: int):
    info = plsc.get_sparse_core_info()
    nw = info.num_cores * info.num_subcores
    assert B == nw * _C

    mesh = plsc.VectorSubcoreMesh(core_axis_name="c", subcore_axis_name="s")
    ngrp = _C // _L

    @functools.partial(
        pl.kernel,
        mesh=mesh,
        out_type=jax.ShapeDtypeStruct((B, S, _D), jnp.float32),
        compiler_params=pltpu.CompilerParams(
            use_tc_tiling_on_sc=True, needs_layout_passes=False
        ),
        scratch_types=[
            pltpu.VMEM((S, _C), jnp.int32),
            pltpu.VMEM((_C,), jnp.int32),
            pltpu.VMEM((_C,), jnp.int32),
            pltpu.VMEM((_C + _L,), jnp.int32),
            pltpu.VMEM((_C + _L,), jnp.int32),
            pltpu.VMEM((_C, _C), jnp.float32),
            pltpu.VMEM((_C, _C), jnp.float32),
            pltpu.VMEM((_C, _D), jnp.float32),
            pltpu.VMEM((_C, _D), jnp.float32),
            pltpu.SemaphoreType.DMA,
            pltpu.SemaphoreType.DMA,
            pltpu.SemaphoreType.DMA,
            pltpu.SemaphoreType.DMA,
        ],
    )
    def k(tokens_hbm, table_hbm, out_hbm, tokbuf, idx0, idx1, par0, par1,
          g0, g1, o0, o1, gsem0, gsem1, osem0, osem1):
        idx = (idx0, idx1)
        par = (par0, par1)
        gbuf = (g0, g1)
        obuf = (o0, o1)
        gsem = (gsem0, gsem1)
        osem = (osem0, osem1)

        wid = lax.axis_index("s") * info.num_cores + lax.axis_index("c")
        col = wid * _C
        pltpu.sync_copy(tokens_hbm.at[:, pl.ds(col, _C)], tokbuf)

        def build(s, slot):
            for g in range(ngrp):
                sl = pl.ds(g * _L, _L)
                t = tokbuf[s, sl]
                idx[slot][sl] = lax.shift_left(
                    lax.shift_right_logical(t, 11), 10) | (t & (_H - 1))
                par[slot][sl] = lax.shift_left(
                    lax.shift_right_logical(t, 10) & 1, 6)

        def gather(slot):
            return pltpu.async_copy(table_hbm.at[idx[slot]], gbuf[slot],
                                    gsem[slot])

        def out_slice(s):
            return out_hbm.at[pl.ds(col, _C), s, :]

        build(0, 0)
        gather(0)

        @pl.loop(0, S // 2)
        def _outer(so):
            for slot in range(2):
                s = so * 2 + slot
                nslot = 1 - slot

                @pl.when(s + 1 < S)
                def _prefetch():
                    build(s + 1, nslot)
                    gather(nslot)

                pltpu.make_async_copy(table_hbm.at[idx[slot]], gbuf[slot],
                                      gsem[slot]).wait()

                @pl.when(s >= 2)
                def _drain():
                    pltpu.make_async_copy(obuf[slot], out_slice(s - 2),
                                          osem[slot]).wait()

                src = gbuf[slot]
                dst = obuf[slot]
                pslot = par[slot]

                @plsc.parallel_loop(0, _C, unroll=4)
                def _select(j):
                    p = pslot[pl.ds(j, _L)][0]
                    for q in range(_D // _L):
                        dst[j, pl.ds(q * _L, _L)] = (
                            src[j, pl.ds(p + q * _L, _L)] * _SCALE)

                pltpu.async_copy(dst, out_slice(s), osem[slot])

        pltpu.make_async_copy(obuf[0], out_slice(S - 2), osem[0]).wait()
        pltpu.make_async_copy(obuf[1], out_slice(S - 1), osem[1]).wait()

    return k


def kernel(tokens, table):
    s0, s1 = tokens.shape
    v, d = table.shape
    assert d == _D
    tokens_t = tokens.T.astype(jnp.int32)
    table2 = _make_relayout(v)(table.T)
    return _make_gather(s1, s0)(tokens_t, table2)


# speedup vs baseline: 1.5724x; 1.3995x over previous
import functools
import math

import numpy as np
import jax
import jax.numpy as jnp
from jax import lax
from jax.experimental import pallas as pl
from jax.experimental.pallas import tpu as pltpu
from jax.experimental.pallas import tpu_sc as plsc

_D = 64
_SCALE = math.sqrt(_D)
_C = 128
_L = 16
_VB = 2048
_H = _VB // 2
_W = 65


def _relayout_block(t_ref, out_ref):
    x = t_ref[...]
    ii = lax.broadcasted_iota(jnp.int32, (_D, 2 * _D), 0)
    jj = lax.broadcasted_iota(jnp.int32, (_D, 2 * _D), 1)
    e1 = (ii == jj).astype(jnp.float32)
    e2 = ((ii + _D) == jj).astype(jnp.float32)
    dims = (((0,), (0,)), ((), ()))
    left = lax.dot_general(x[:, :_H], e1, dims,
                           preferred_element_type=jnp.float32)
    right = lax.dot_general(x[:, _H:], e2, dims,
                            preferred_element_type=jnp.float32)
    out_ref[...] = left + right


@functools.lru_cache(maxsize=None)
def _make_relayout(v: int):
    nb = (v + _VB - 1) // _VB
    return pl.pallas_call(
        _relayout_block,
        grid=(nb,),
        in_specs=[pl.BlockSpec((_D, _VB), lambda i: (0, i))],
        out_specs=pl.BlockSpec((_H, 2 * _D), lambda i: (i, 0)),
        out_shape=jax.ShapeDtypeStruct((nb * _H, 2 * _D), jnp.float32),
    )


@functools.lru_cache(maxsize=None)
def _make_gather(S: int, B: int):
    info = plsc.get_sparse_core_info()
    nw = info.num_cores * info.num_subcores
    assert B == nw * _C

    mesh = plsc.VectorSubcoreMesh(core_axis_name="c", subcore_axis_name="s")
    ngrp = _C // _L

    @functools.partial(
        pl.kernel,
        mesh=mesh,
        out_type=jax.ShapeDtypeStruct((S, _D, B), jnp.float32),
        compiler_params=pltpu.CompilerParams(
            use_tc_tiling_on_sc=True, needs_layout_passes=False
        ),
        scratch_types=[
            pltpu.VMEM((S, _C), jnp.int32),
            pltpu.VMEM((_C,), jnp.int32),
            pltpu.VMEM((_C,), jnp.int32),
            pltpu.VMEM((_C + _L,), jnp.int32),
            pltpu.VMEM((_C + _L,), jnp.int32),
            pltpu.VMEM((_C, _C), jnp.float32),
            pltpu.VMEM((_C, _C), jnp.float32),
            pltpu.VMEM((_C * _W,), jnp.float32),
            pltpu.VMEM((_D, _C), jnp.float32),
            pltpu.VMEM((_D, _C), jnp.float32),
            pltpu.SemaphoreType.DMA,
            pltpu.SemaphoreType.DMA,
            pltpu.SemaphoreType.DMA,
            pltpu.SemaphoreType.DMA,
        ],
    )
    def k(tokens_hbm, table_hbm, out_hbm, tokbuf, idx0, idx1, par0, par1,
          g0, g1, sbuf, o0, o1, gsem0, gsem1, osem0, osem1):
        idx = (idx0, idx1)
        par = (par0, par1)
        gbuf = (g0, g1)
        obuf = (o0, o1)
        gsem = (gsem0, gsem1)
        osem = (osem0, osem1)

        wid = lax.axis_index("s") * info.num_cores + lax.axis_index("c")
        col = wid * _C
        pltpu.sync_copy(tokens_hbm.at[:, pl.ds(col, _C)], tokbuf)

        def build(s, slot):
            for g in range(ngrp):
                sl = pl.ds(g * _L, _L)
                t = tokbuf[s, sl]
                idx[slot][sl] = lax.shift_left(
                    lax.shift_right_logical(t, 11), 10) | (t & (_H - 1))
                par[slot][sl] = lax.shift_left(
                    lax.shift_right_logical(t, 10) & 1, 6)

        def gather(slot):
            return pltpu.async_copy(table_hbm.at[idx[slot]], gbuf[slot],
                                    gsem[slot])

        def out_slice(s):
            return out_hbm.at[s, :, pl.ds(col, _C)]

        row65 = [(lax.iota(jnp.int32, _L) + g * _L) * _W for g in range(ngrp)]

        build(0, 0)
        gather(0)

        @pl.loop(0, S // 2)
        def _outer(so):
            for slot in range(2):
                s = so * 2 + slot
                nslot = 1 - slot

                @pl.when(s + 1 < S)
                def _prefetch():
                    build(s + 1, nslot)
                    gather(nslot)

                pltpu.make_async_copy(table_hbm.at[idx[slot]], gbuf[slot],
                                      gsem[slot]).wait()

                @pl.when(s >= 2)
                def _drain():
                    pltpu.make_async_copy(obuf[slot], out_slice(s - 2),
                                          osem[slot]).wait()

                src = gbuf[slot]
                dst = obuf[slot]
                pslot = par[slot]

                @plsc.parallel_loop(0, _C, unroll=4)
                def _select(j):
                    p = pslot[pl.ds(j, _L)][0]
                    base = j * _W
                    for q in range(_D // _L):
                        sbuf[pl.ds(base + q * _L, _L)] = (
                            src[j, pl.ds(p + q * _L, _L)] * _SCALE)

                @plsc.parallel_loop(0, _D, unroll=2)
                def _transpose(f):
                    for g in range(ngrp):
                        dst[f, pl.ds(g * _L, _L)] = plsc.load_gather(
                            sbuf, [row65[g] + f])

                pltpu.async_copy(dst, out_slice(s), osem[slot])

        pltpu.make_async_copy(obuf[0], out_slice(S - 2), osem[0]).wait()
        pltpu.make_async_copy(obuf[1], out_slice(S - 1), osem[1]).wait()

    return k


def kernel(tokens, table):
    s0, s1 = tokens.shape
    v, d = table.shape
    assert d == _D
    tokens_t = tokens.T.astype(jnp.int32)
    table2 = _make_relayout(v)(table.T)
    out = _make_gather(s1, s0)(tokens_t, table2)
    return jnp.transpose(out, (2, 0, 1))


# speedup vs baseline: 1.9113x; 1.2155x over previous
import functools
import math

import numpy as np
import jax
import jax.numpy as jnp
from jax import lax
from jax.experimental import pallas as pl
from jax.experimental.pallas import tpu as pltpu
from jax.experimental.pallas import tpu_sc as plsc

_D = 64
_SCALE = math.sqrt(_D)
_C = 128
_L = 16
_VB = 4096
_H = _VB // 2
_W = 65


def _relayout_block(t_ref, out_ref):
    x = t_ref[...]
    ii = lax.broadcasted_iota(jnp.int32, (_D, 2 * _D), 0)
    jj = lax.broadcasted_iota(jnp.int32, (_D, 2 * _D), 1)
    e1 = (ii == jj).astype(jnp.float32)
    e2 = ((ii + _D) == jj).astype(jnp.float32)
    dims = (((0,), (0,)), ((), ()))
    left = lax.dot_general(x[:, :_H], e1, dims,
                           preferred_element_type=jnp.float32)
    right = lax.dot_general(x[:, _H:], e2, dims,
                            preferred_element_type=jnp.float32)
    out_ref[...] = left + right


@functools.lru_cache(maxsize=None)
def _make_relayout(v: int):
    nb = (v + _VB - 1) // _VB
    return pl.pallas_call(
        _relayout_block,
        grid=(nb,),
        in_specs=[pl.BlockSpec((_D, _VB), lambda i: (0, i))],
        out_specs=pl.BlockSpec((_H, 2 * _D), lambda i: (i, 0)),
        out_shape=jax.ShapeDtypeStruct((nb * _H, 2 * _D), jnp.float32),
    )


@functools.lru_cache(maxsize=None)
def _make_gather(S: int, B: int):
    info = plsc.get_sparse_core_info()
    nw = info.num_cores * info.num_subcores
    assert B == nw * _C

    mesh = plsc.VectorSubcoreMesh(core_axis_name="c", subcore_axis_name="s")
    ngrp = _C // _L

    @functools.partial(
        pl.kernel,
        mesh=mesh,
        out_type=jax.ShapeDtypeStruct((S, _D, B), jnp.float32),
        compiler_params=pltpu.CompilerParams(
            use_tc_tiling_on_sc=True, needs_layout_passes=False
        ),
        scratch_types=[
            pltpu.VMEM((S, _C), jnp.int32),
            pltpu.VMEM((_C,), jnp.int32),
            pltpu.VMEM((_C,), jnp.int32),
            pltpu.VMEM((_C + _L,), jnp.int32),
            pltpu.VMEM((_C + _L,), jnp.int32),
            pltpu.VMEM((_C, _C), jnp.float32),
            pltpu.VMEM((_C, _C), jnp.float32),
            pltpu.VMEM((_C * _W,), jnp.float32),
            pltpu.VMEM((_D, _C), jnp.float32),
            pltpu.VMEM((_D, _C), jnp.float32),
            pltpu.SemaphoreType.DMA,
            pltpu.SemaphoreType.DMA,
            pltpu.SemaphoreType.DMA,
            pltpu.SemaphoreType.DMA,
        ],
    )
    def k(tokens_hbm, table_hbm, out_hbm, tokbuf, idx0, idx1, par0, par1,
          g0, g1, sbuf, o0, o1, gsem0, gsem1, osem0, osem1):
        idx = (idx0, idx1)
        par = (par0, par1)
        gbuf = (g0, g1)
        obuf = (o0, o1)
        gsem = (gsem0, gsem1)
        osem = (osem0, osem1)

        wid = lax.axis_index("s") * info.num_cores + lax.axis_index("c")
        col = wid * _C
        pltpu.sync_copy(tokens_hbm.at[:, pl.ds(col, _C)], tokbuf)

        shv = _VB.bit_length() - 1
        shh = shv - 1

        def build(s, slot):
            for g in range(ngrp):
                sl = pl.ds(g * _L, _L)
                t = tokbuf[s, sl]
                idx[slot][sl] = lax.shift_left(
                    lax.shift_right_logical(t, shv), shh) | (t & (_H - 1))
                par[slot][sl] = lax.shift_left(
                    lax.shift_right_logical(t, shh) & 1, 6)

        def gather(slot):
            return pltpu.async_copy(table_hbm.at[idx[slot]], gbuf[slot],
                                    gsem[slot])

        def out_slice(s):
            return out_hbm.at[s, :, pl.ds(col, _C)]

        row65 = [(lax.iota(jnp.int32, _L) + g * _L) * _W for g in range(ngrp)]

        build(0, 0)
        gather(0)

        @pl.loop(0, S // 2)
        def _outer(so):
            for slot in range(2):
                s = so * 2 + slot
                nslot = 1 - slot

                @pl.when(s + 1 < S)
                def _prefetch():
                    build(s + 1, nslot)
                    gather(nslot)

                pltpu.make_async_copy(table_hbm.at[idx[slot]], gbuf[slot],
                                      gsem[slot]).wait()

                @pl.when(s >= 2)
                def _drain():
                    pltpu.make_async_copy(obuf[slot], out_slice(s - 2),
                                          osem[slot]).wait()

                src = gbuf[slot]
                dst = obuf[slot]
                pslot = par[slot]

                @plsc.parallel_loop(0, _C, unroll=4)
                def _select(j):
                    p = pslot[pl.ds(j, _L)][0]
                    base = j * _W
                    for q in range(_D // _L):
                        sbuf[pl.ds(base + q * _L, _L)] = (
                            src[j, pl.ds(p + q * _L, _L)] * _SCALE)

                @plsc.parallel_loop(0, _D, unroll=2)
                def _transpose(f):
                    for g in range(ngrp):
                        dst[f, pl.ds(g * _L, _L)] = plsc.load_gather(
                            sbuf, [row65[g] + f])

                pltpu.async_copy(dst, out_slice(s), osem[slot])

        pltpu.make_async_copy(obuf[0], out_slice(S - 2), osem[0]).wait()
        pltpu.make_async_copy(obuf[1], out_slice(S - 1), osem[1]).wait()

    return k


def kernel(tokens, table):
    s0, s1 = tokens.shape
    v, d = table.shape
    assert d == _D
    tokens_t = tokens.T.astype(jnp.int32)
    table2 = _make_relayout(v)(table.T)
    out = _make_gather(s1, s0)(tokens_t, table2)
    return jnp.transpose(out, (2, 0, 1))


# speedup vs baseline: 2.1816x; 1.1415x over previous
import functools
import math

import numpy as np
import jax
import jax.numpy as jnp
from jax import lax
from jax.experimental import pallas as pl
from jax.experimental.pallas import tpu as pltpu
from jax.experimental.pallas import tpu_sc as plsc

_D = 64
_SCALE = math.sqrt(_D)
_C = 128
_L = 16
_VB = 8192
_H = _VB // 2
_W = 65


def _relayout_block(t_ref, out_ref):
    x = t_ref[...]
    ii = lax.broadcasted_iota(jnp.int32, (_D, 2 * _D), 0)
    jj = lax.broadcasted_iota(jnp.int32, (_D, 2 * _D), 1)
    e1 = (ii == jj).astype(jnp.float32)
    e2 = ((ii + _D) == jj).astype(jnp.float32)
    dims = (((0,), (0,)), ((), ()))
    left = lax.dot_general(x[:, :_H], e1, dims,
                           preferred_element_type=jnp.float32)
    right = lax.dot_general(x[:, _H:], e2, dims,
                            preferred_element_type=jnp.float32)
    out_ref[...] = left + right


@functools.lru_cache(maxsize=None)
def _make_relayout(v: int):
    nb = (v + _VB - 1) // _VB
    return pl.pallas_call(
        _relayout_block,
        grid=(nb,),
        in_specs=[pl.BlockSpec((_D, _VB), lambda i: (0, i))],
        out_specs=pl.BlockSpec((_H, 2 * _D), lambda i: (i, 0)),
        out_shape=jax.ShapeDtypeStruct((nb * _H, 2 * _D), jnp.float32),
    )


@functools.lru_cache(maxsize=None)
def _make_gather(S: int, B: int):
    info = plsc.get_sparse_core_info()
    nw = info.num_cores * info.num_subcores
    assert B == nw * _C

    mesh = plsc.VectorSubcoreMesh(core_axis_name="c", subcore_axis_name="s")
    ngrp = _C // _L

    @functools.partial(
        pl.kernel,
        mesh=mesh,
        out_type=jax.ShapeDtypeStruct((S, _D, B), jnp.float32),
        compiler_params=pltpu.CompilerParams(
            use_tc_tiling_on_sc=True, needs_layout_passes=False
        ),
        scratch_types=[
            pltpu.VMEM((S, _C), jnp.int32),
            pltpu.VMEM((_C,), jnp.int32),
            pltpu.VMEM((_C,), jnp.int32),
            pltpu.VMEM((_C + _L,), jnp.int32),
            pltpu.VMEM((_C + _L,), jnp.int32),
            pltpu.VMEM((_C, _C), jnp.float32),
            pltpu.VMEM((_C, _C), jnp.float32),
            pltpu.VMEM((_C * _W,), jnp.float32),
            pltpu.VMEM((_D, _C), jnp.float32),
            pltpu.VMEM((_D, _C), jnp.float32),
            pltpu.SemaphoreType.DMA,
            pltpu.SemaphoreType.DMA,
            pltpu.SemaphoreType.DMA,
            pltpu.SemaphoreType.DMA,
        ],
    )
    def k(tokens_hbm, table_hbm, out_hbm, tokbuf, idx0, idx1, par0, par1,
          g0, g1, sbuf, o0, o1, gsem0, gsem1, osem0, osem1):
        idx = (idx0, idx1)
        par = (par0, par1)
        gbuf = (g0, g1)
        obuf = (o0, o1)
        gsem = (gsem0, gsem1)
        osem = (osem0, osem1)

        wid = lax.axis_index("s") * info.num_cores + lax.axis_index("c")
        col = wid * _C
        pltpu.sync_copy(tokens_hbm.at[:, pl.ds(col, _C)], tokbuf)

        shv = _VB.bit_length() - 1
        shh = shv - 1

        def build(s, slot):
            for g in range(ngrp):
                sl = pl.ds(g * _L, _L)
                t = tokbuf[s, sl]
                idx[slot][sl] = lax.shift_left(
                    lax.shift_right_logical(t, shv), shh) | (t & (_H - 1))
                par[slot][sl] = lax.shift_left(
                    lax.shift_right_logical(t, shh) & 1, 6)

        def gather(slot):
            return pltpu.async_copy(table_hbm.at[idx[slot]], gbuf[slot],
                                    gsem[slot])

        def out_slice(s):
            return out_hbm.at[s, :, pl.ds(col, _C)]

        row65 = [(lax.iota(jnp.int32, _L) + g * _L) * _W for g in range(ngrp)]

        build(0, 0)
        gather(0)

        @pl.loop(0, S // 2)
        def _outer(so):
            for slot in range(2):
                s = so * 2 + slot
                nslot = 1 - slot

                @pl.when(s + 1 < S)
                def _prefetch():
                    build(s + 1, nslot)
                    gather(nslot)

                pltpu.make_async_copy(table_hbm.at[idx[slot]], gbuf[slot],
                                      gsem[slot]).wait()

                @pl.when(s >= 2)
                def _drain():
                    pltpu.make_async_copy(obuf[slot], out_slice(s - 2),
                                          osem[slot]).wait()

                src = gbuf[slot]
                dst = obuf[slot]
                pslot = par[slot]

                @plsc.parallel_loop(0, _C, unroll=4)
                def _select(j):
                    p = pslot[pl.ds(j, _L)][0]
                    base = j * _W
                    for q in range(_D // _L):
                        sbuf[pl.ds(base + q * _L, _L)] = (
                            src[j, pl.ds(p + q * _L, _L)] * _SCALE)

                @plsc.parallel_loop(0, _D, unroll=2)
                def _transpose(f):
                    for g in range(ngrp):
                        dst[f, pl.ds(g * _L, _L)] = plsc.load_gather(
                            sbuf, [row65[g] + f])

                pltpu.async_copy(dst, out_slice(s), osem[slot])

        pltpu.make_async_copy(obuf[0], out_slice(S - 2), osem[0]).wait()
        pltpu.make_async_copy(obuf[1], out_slice(S - 1), osem[1]).wait()

    return k


def kernel(tokens, table):
    s0, s1 = tokens.shape
    v, d = table.shape
    assert d == _D
    tokens_t = tokens.T.astype(jnp.int32)
    table2 = _make_relayout(v)(table.T)
    out = _make_gather(s1, s0)(tokens_t, table2)
    return jnp.transpose(out, (2, 0, 1))


# speedup vs baseline: 2.3405x; 1.0728x over previous
import functools
import math

import numpy as np
import jax
import jax.numpy as jnp
from jax import lax
from jax.experimental import pallas as pl
from jax.experimental.pallas import tpu as pltpu
from jax.experimental.pallas import tpu_sc as plsc

_D = 64
_SCALE = math.sqrt(_D)
_C = 128
_L = 16
_VB = 16384
_H = _VB // 2
_W = 65


def _relayout_block(t_ref, out_ref):
    x = t_ref[...]
    ii = lax.broadcasted_iota(jnp.int32, (_D, 2 * _D), 0)
    jj = lax.broadcasted_iota(jnp.int32, (_D, 2 * _D), 1)
    e1 = (ii == jj).astype(jnp.float32)
    e2 = ((ii + _D) == jj).astype(jnp.float32)
    dims = (((0,), (0,)), ((), ()))
    left = lax.dot_general(x[:, :_H], e1, dims,
                           preferred_element_type=jnp.float32)
    right = lax.dot_general(x[:, _H:], e2, dims,
                            preferred_element_type=jnp.float32)
    out_ref[...] = left + right


@functools.lru_cache(maxsize=None)
def _make_relayout(v: int):
    nb = (v + _VB - 1) // _VB
    return pl.pallas_call(
        _relayout_block,
        grid=(nb,),
        in_specs=[pl.BlockSpec((_D, _VB), lambda i: (0, i))],
        out_specs=pl.BlockSpec((_H, 2 * _D), lambda i: (i, 0)),
        out_shape=jax.ShapeDtypeStruct((nb * _H, 2 * _D), jnp.float32),
    )


@functools.lru_cache(maxsize=None)
def _make_gather(S: int, B: int):
    info = plsc.get_sparse_core_info()
    nw = info.num_cores * info.num_subcores
    assert B == nw * _C

    mesh = plsc.VectorSubcoreMesh(core_axis_name="c", subcore_axis_name="s")
    ngrp = _C // _L

    @functools.partial(
        pl.kernel,
        mesh=mesh,
        out_type=jax.ShapeDtypeStruct((S, _D, B), jnp.float32),
        compiler_params=pltpu.CompilerParams(
            use_tc_tiling_on_sc=True, needs_layout_passes=False
        ),
        scratch_types=[
            pltpu.VMEM((S, _C), jnp.int32),
            pltpu.VMEM((_C,), jnp.int32),
            pltpu.VMEM((_C,), jnp.int32),
            pltpu.VMEM((_C + _L,), jnp.int32),
            pltpu.VMEM((_C + _L,), jnp.int32),
            pltpu.VMEM((_C, _C), jnp.float32),
            pltpu.VMEM((_C, _C), jnp.float32),
            pltpu.VMEM((_C * _W,), jnp.float32),
            pltpu.VMEM((_D, _C), jnp.float32),
            pltpu.VMEM((_D, _C), jnp.float32),
            pltpu.SemaphoreType.DMA,
            pltpu.SemaphoreType.DMA,
            pltpu.SemaphoreType.DMA,
            pltpu.SemaphoreType.DMA,
        ],
    )
    def k(tokens_hbm, table_hbm, out_hbm, tokbuf, idx0, idx1, par0, par1,
          g0, g1, sbuf, o0, o1, gsem0, gsem1, osem0, osem1):
        idx = (idx0, idx1)
        par = (par0, par1)
        gbuf = (g0, g1)
        obuf = (o0, o1)
        gsem = (gsem0, gsem1)
        osem = (osem0, osem1)

        wid = lax.axis_index("s") * info.num_cores + lax.axis_index("c")
        col = wid * _C
        pltpu.sync_copy(tokens_hbm.at[:, pl.ds(col, _C)], tokbuf)

        shv = _VB.bit_length() - 1
        shh = shv - 1

        def build(s, slot):
            for g in range(ngrp):
                sl = pl.ds(g * _L, _L)
                t = tokbuf[s, sl]
                idx[slot][sl] = lax.shift_left(
                    lax.shift_right_logical(t, shv), shh) | (t & (_H - 1))
                par[slot][sl] = lax.shift_left(
                    lax.shift_right_logical(t, shh) & 1, 6)

        def gather(slot):
            return pltpu.async_copy(table_hbm.at[idx[slot]], gbuf[slot],
                                    gsem[slot])

        def out_slice(s):
            return out_hbm.at[s, :, pl.ds(col, _C)]

        row65 = [(lax.iota(jnp.int32, _L) + g * _L) * _W for g in range(ngrp)]

        build(0, 0)
        gather(0)

        @pl.loop(0, S // 2)
        def _outer(so):
            for slot in range(2):
                s = so * 2 + slot
                nslot = 1 - slot

                @pl.when(s + 1 < S)
                def _prefetch():
                    build(s + 1, nslot)
                    gather(nslot)

                pltpu.make_async_copy(table_hbm.at[idx[slot]], gbuf[slot],
                                      gsem[slot]).wait()

                @pl.when(s >= 2)
                def _drain():
                    pltpu.make_async_copy(obuf[slot], out_slice(s - 2),
                                          osem[slot]).wait()

                src = gbuf[slot]
                dst = obuf[slot]
                pslot = par[slot]

                @plsc.parallel_loop(0, _C, unroll=4)
                def _select(j):
                    p = pslot[pl.ds(j, _L)][0]
                    base = j * _W
                    for q in range(_D // _L):
                        sbuf[pl.ds(base + q * _L, _L)] = (
                            src[j, pl.ds(p + q * _L, _L)] * _SCALE)

                @plsc.parallel_loop(0, _D, unroll=2)
                def _transpose(f):
                    for g in range(ngrp):
                        dst[f, pl.ds(g * _L, _L)] = plsc.load_gather(
                            sbuf, [row65[g] + f])

                pltpu.async_copy(dst, out_slice(s), osem[slot])

        pltpu.make_async_copy(obuf[0], out_slice(S - 2), osem[0]).wait()
        pltpu.make_async_copy(obuf[1], out_slice(S - 1), osem[1]).wait()

    return k


def kernel(tokens, table):
    s0, s1 = tokens.shape
    v, d = table.shape
    assert d == _D
    tokens_t = tokens.T.astype(jnp.int32)
    table2 = _make_relayout(v)(table.T)
    out = _make_gather(s1, s0)(tokens_t, table2)
    return jnp.transpose(out, (2, 0, 1))
